# baseline (speedup 1.0000x reference)
import functools

import jax
import jax.numpy as jnp
from jax.experimental import pallas as pl
from jax.experimental.pallas import tpu as pltpu
from jax.experimental.pallas import tpu_sc as plsc

BATCH = 16384
EMBED = 64
WINDOW = 128


def _sc_gather(table2, idx):
    num_windows = BATCH // WINDOW
    mesh = plsc.VectorSubcoreMesh(core_axis_name="core",
                                  subcore_axis_name="subcore")

    @functools.partial(
        pl.kernel,
        out_type=jax.ShapeDtypeStruct((BATCH, 2 * EMBED), table2.dtype),
        mesh=mesh,
        compiler_params=pltpu.CompilerParams(use_tc_tiling_on_sc=True),
    )
    def gather_kernel(x_hbm, i_hbm, o_hbm):
        def body(i_vmem, o_vmem):
            pltpu.sync_copy(x_hbm.at[i_vmem.at[0]], o_vmem)

        pltpu.emit_pipeline(
            body,
            grid=(num_windows,),
            in_specs=[pl.BlockSpec((1, WINDOW), index_map=lambda i: (0, i))],
            out_specs=[pl.BlockSpec((WINDOW, 2 * EMBED),
                                    index_map=lambda i: (i, 0))],
            core_axis_name=("core", "subcore"),
            dimension_semantics=(pltpu.PARALLEL,),
        )(i_hbm, o_hbm)

    return gather_kernel(table2, idx.reshape(1, BATCH))


TC_BLOCK = 2048


def _tc_loss_body(f_ref, g_ref, lab_ref, o_ref):
    f = f_ref[...]
    g = g_ref[...]
    par = lab_ref[...] % 2
    c = jnp.where(par == 0, g[:, :EMBED], g[:, EMBED:])
    fn = jnp.sqrt(jnp.sum(f * f, axis=1, keepdims=True))
    f1 = f / jnp.maximum(fn, 1e-12)
    cn = jnp.sqrt(jnp.sum(c * c, axis=1, keepdims=True))
    c1 = c / jnp.maximum(cn, 1e-12)
    num = jnp.sum(f1 * c1, axis=1, keepdims=True)
    d1 = jnp.sqrt(jnp.sum(f1 * f1, axis=1, keepdims=True))
    d2 = jnp.sqrt(jnp.sum(c1 * c1, axis=1, keepdims=True))
    cos = num / jnp.maximum(d1 * d2, 1e-8)
    part = jnp.sum(1.0 - cos, axis=0, keepdims=True) / BATCH

    @pl.when(pl.program_id(0) == 0)
    def _():
        o_ref[...] = jnp.zeros_like(o_ref)

    o_ref[...] += part


def _tc_loss(features, gathered, labels_2d):
    return pl.pallas_call(
        _tc_loss_body,
        grid=(BATCH // TC_BLOCK,),
        in_specs=[
            pl.BlockSpec((TC_BLOCK, EMBED), lambda i: (i, 0)),
            pl.BlockSpec((TC_BLOCK, 2 * EMBED), lambda i: (i, 0)),
            pl.BlockSpec((TC_BLOCK, 1), lambda i: (i, 0)),
        ],
        out_specs=pl.BlockSpec((1, 1), lambda i: (0, 0)),
        out_shape=jax.ShapeDtypeStruct((1, 1), jnp.float32),
    )(features, gathered, labels_2d)


def kernel(features, labels, centers):
    labels32 = labels.astype(jnp.int32)
    table2 = centers.reshape(centers.shape[0] // 2, 2 * EMBED)
    gathered = _sc_gather(table2, labels32 // 2)
    loss = _tc_loss(features, gathered, labels32.reshape(BATCH, 1))
    return loss[0, 0]


# speedup vs baseline: 1.1054x; 1.1054x over previous
import functools

import jax
import jax.numpy as jnp
from jax.experimental import pallas as pl
from jax.experimental.pallas import tpu as pltpu
from jax.experimental.pallas import tpu_sc as plsc

BATCH = 16384
EMBED = 64
NUM_CLASSES = 1000000
PACK_W = 4096
PACK_H = PACK_W // 2
NUM_WINDOWS = (NUM_CLASSES + PACK_W - 1) // PACK_W
TABLE_ROWS = NUM_WINDOWS * PACK_H
GATHER_WINDOW = 128


def _tc_pack_body(x_ref, o_ref):
    eye = jnp.float32(1.0) * (
        jax.lax.broadcasted_iota(jnp.int32, (EMBED, EMBED), 0)
        == jax.lax.broadcasted_iota(jnp.int32, (EMBED, EMBED), 1))
    dn = (((0,), (0,)), ((), ()))
    t0 = jax.lax.dot_general(x_ref[:, :PACK_H], eye, dn,
                             precision=jax.lax.Precision.HIGHEST,
                             preferred_element_type=jnp.float32)
    t1 = jax.lax.dot_general(x_ref[:, PACK_H:], eye, dn,
                             precision=jax.lax.Precision.HIGHEST,
                             preferred_element_type=jnp.float32)
    o_ref[...] = jnp.concatenate([t0, t1], axis=1)


def _tc_pack(centers_t):
    return pl.pallas_call(
        _tc_pack_body,
        grid=(NUM_WINDOWS,),
        in_specs=[pl.BlockSpec((EMBED, PACK_W), lambda b: (0, b))],
        out_specs=pl.BlockSpec((PACK_H, 2 * EMBED), lambda b: (b, 0)),
        out_shape=jax.ShapeDtypeStruct((TABLE_ROWS, 2 * EMBED), jnp.float32),
    )(centers_t)


def _sc_gather(table2, idx):
    num_steps = BATCH // GATHER_WINDOW
    mesh = plsc.VectorSubcoreMesh(core_axis_name="core",
                                  subcore_axis_name="subcore")

    @functools.partial(
        pl.kernel,
        out_type=jax.ShapeDtypeStruct((BATCH, 2 * EMBED), table2.dtype),
        mesh=mesh,
        compiler_params=pltpu.CompilerParams(use_tc_tiling_on_sc=True),
    )
    def gather_kernel(x_hbm, i_hbm, o_hbm):
        def body(i_vmem, o_vmem):
            pltpu.sync_copy(x_hbm.at[i_vmem.at[0]], o_vmem)

        pltpu.emit_pipeline(
            body,
            grid=(num_steps,),
            in_specs=[pl.BlockSpec((1, GATHER_WINDOW),
                                   index_map=lambda i: (0, i))],
            out_specs=[pl.BlockSpec((GATHER_WINDOW, 2 * EMBED),
                                    index_map=lambda i: (i, 0))],
            core_axis_name=("core", "subcore"),
            dimension_semantics=(pltpu.PARALLEL,),
        )(i_hbm, o_hbm)

    return gather_kernel(table2, idx.reshape(1, BATCH))


TC_BLOCK = 2048


def _tc_loss_body(f_ref, g_ref, half_ref, o_ref):
    f = f_ref[...]
    g = g_ref[...]
    sel = half_ref[...] != 0
    c = jnp.where(sel, g[:, EMBED:], g[:, :EMBED])
    fn = jnp.sqrt(jnp.sum(f * f, axis=1, keepdims=True))
    f1 = f / jnp.maximum(fn, 1e-12)
    cn = jnp.sqrt(jnp.sum(c * c, axis=1, keepdims=True))
    c1 = c / jnp.maximum(cn, 1e-12)
    num = jnp.sum(f1 * c1, axis=1, keepdims=True)
    d1 = jnp.sqrt(jnp.sum(f1 * f1, axis=1, keepdims=True))
    d2 = jnp.sqrt(jnp.sum(c1 * c1, axis=1, keepdims=True))
    cos = num / jnp.maximum(d1 * d2, 1e-8)
    part = jnp.sum(1.0 - cos, axis=0, keepdims=True) / BATCH

    @pl.when(pl.program_id(0) == 0)
    def _():
        o_ref[...] = jnp.zeros_like(o_ref)

    o_ref[...] += part


def _tc_loss(features, gathered, half_2d):
    return pl.pallas_call(
        _tc_loss_body,
        grid=(BATCH // TC_BLOCK,),
        in_specs=[
            pl.BlockSpec((TC_BLOCK, EMBED), lambda i: (i, 0)),
            pl.BlockSpec((TC_BLOCK, 2 * EMBED), lambda i: (i, 0)),
            pl.BlockSpec((TC_BLOCK, 1), lambda i: (i, 0)),
        ],
        out_specs=pl.BlockSpec((1, 1), lambda i: (0, 0)),
        out_shape=jax.ShapeDtypeStruct((1, 1), jnp.float32),
    )(features, gathered, half_2d)


def kernel(features, labels, centers):
    labels32 = labels.astype(jnp.int32)
    table2 = _tc_pack(centers.T)
    w = labels32 // PACK_W
    r = labels32 % PACK_W
    slot = w * PACK_H + r % PACK_H
    half = r // PACK_H
    gathered = _sc_gather(table2, slot)
    loss = _tc_loss(features, gathered, half.reshape(BATCH, 1))
    return loss[0, 0]


# speedup vs baseline: 2.6097x; 2.3609x over previous
import functools

import jax
import jax.numpy as jnp
from jax.experimental import pallas as pl
from jax.experimental.pallas import tpu as pltpu
from jax.experimental.pallas import tpu_sc as plsc

BATCH = 16384
EMBED = 64
NUM_CLASSES = 1000000
QUARTERS = 4
PACK_H = 2048
PACK_W = QUARTERS * PACK_H
NUM_WINDOWS = (NUM_CLASSES + PACK_W - 1) // PACK_W
TABLE_ROWS = NUM_WINDOWS * PACK_H
ROW_W = QUARTERS * EMBED
GATHER_WINDOW = 128


def _tc_pack_body(x_ref, o_ref):
    n = QUARTERS * EMBED
    eye = jnp.float32(1.0) * (
        jax.lax.broadcasted_iota(jnp.int32, (n, n), 0)
        == jax.lax.broadcasted_iota(jnp.int32, (n, n), 1))
    xr = jnp.concatenate(
        [x_ref[:, q * PACK_H:(q + 1) * PACK_H] for q in range(QUARTERS)],
        axis=0)
    o_ref[...] = jax.lax.dot_general(
        xr, eye, (((0,), (0,)), ((), ())),
        preferred_element_type=jnp.float32)


def _tc_pack(centers_t):
    return pl.pallas_call(
        _tc_pack_body,
        grid=(NUM_WINDOWS,),
        in_specs=[pl.BlockSpec((EMBED, PACK_W), lambda b: (0, b))],
        out_specs=pl.BlockSpec((PACK_H, ROW_W), lambda b: (b, 0)),
        out_shape=jax.ShapeDtypeStruct((TABLE_ROWS, ROW_W), jnp.float32),
    )(centers_t)


def _sc_gather(table, idx):
    num_steps = BATCH // GATHER_WINDOW
    mesh = plsc.VectorSubcoreMesh(core_axis_name="core",
                                  subcore_axis_name="subcore")

    @functools.partial(
        pl.kernel,
        out_type=jax.ShapeDtypeStruct((BATCH, ROW_W), table.dtype),
        mesh=mesh,
        compiler_params=pltpu.CompilerParams(use_tc_tiling_on_sc=True),
    )
    def gather_kernel(x_hbm, i_hbm, o_hbm):
        def body(i_vmem, o_vmem):
            pltpu.sync_copy(x_hbm.at[i_vmem.at[0]], o_vmem)

        pltpu.emit_pipeline(
            body,
            grid=(num_steps,),
            in_specs=[pl.BlockSpec((1, GATHER_WINDOW),
                                   index_map=lambda i: (0, i))],
            out_specs=[pl.BlockSpec((GATHER_WINDOW, ROW_W),
                                    index_map=lambda i: (i, 0))],
            core_axis_name=("core", "subcore"),
            dimension_semantics=(pltpu.PARALLEL,),
        )(i_hbm, o_hbm)

    return gather_kernel(table, idx.reshape(1, BATCH))


TC_BLOCK = 2048


def _tc_loss_body(f_ref, g_ref, q_ref, o_ref):
    f = f_ref[...]
    g = g_ref[...]
    q = q_ref[...]
    c01 = jnp.where(q == 0, g[:, :EMBED], g[:, EMBED:2 * EMBED])
    c23 = jnp.where(q == 2, g[:, 2 * EMBED:3 * EMBED], g[:, 3 * EMBED:])
    c = jnp.where(q < 2, c01, c23)
    num = jnp.sum(f * c, axis=1, keepdims=True)
    ff = jnp.sum(f * f, axis=1, keepdims=True)
    cc = jnp.sum(c * c, axis=1, keepdims=True)
    cos = num * jax.lax.rsqrt(jnp.maximum(ff * cc, 1e-16))
    part = jnp.sum(1.0 - cos, axis=0, keepdims=True) / BATCH

    @pl.when(pl.program_id(0) == 0)
    def _():
        o_ref[...] = jnp.zeros_like(o_ref)

    o_ref[...] += part


def _tc_loss(features, gathered, quarter_2d):
    return pl.pallas_call(
        _tc_loss_body,
        grid=(BATCH // TC_BLOCK,),
        in_specs=[
            pl.BlockSpec((TC_BLOCK, EMBED), lambda i: (i, 0)),
            pl.BlockSpec((TC_BLOCK, ROW_W), lambda i: (i, 0)),
            pl.BlockSpec((TC_BLOCK, 1), lambda i: (i, 0)),
        ],
        out_specs=pl.BlockSpec((1, 1), lambda i: (0, 0)),
        out_shape=jax.ShapeDtypeStruct((1, 1), jnp.float32),
    )(features, gathered, quarter_2d)


def kernel(features, labels, centers):
    labels32 = labels.astype(jnp.int32)
    table = _tc_pack(centers.T)
    w = labels32 // PACK_W
    r = labels32 % PACK_W
    slot = w * PACK_H + r % PACK_H
    quarter = r // PACK_H
    gathered = _sc_gather(table, slot)
    loss = _tc_loss(features, gathered, quarter.reshape(BATCH, 1))
    return loss[0, 0]


# speedup vs baseline: 2.9512x; 1.1308x over previous
import functools

import jax
import jax.numpy as jnp
from jax.experimental import pallas as pl
from jax.experimental.pallas import tpu as pltpu
from jax.experimental.pallas import tpu_sc as plsc

BATCH = 16384
EMBED = 64
NUM_CLASSES = 1000000
QUARTERS = 4
PACK_H = 4096
PACK_W = QUARTERS * PACK_H
NUM_WINDOWS = (NUM_CLASSES + PACK_W - 1) // PACK_W
TABLE_ROWS = NUM_WINDOWS * PACK_H
ROW_W = QUARTERS * EMBED
GATHER_WINDOW = 128


def _tc_pack_body(x_ref, o_ref):
    n = QUARTERS * EMBED
    eye = jnp.bfloat16(1.0) * (
        jax.lax.broadcasted_iota(jnp.int32, (n, n), 0)
        == jax.lax.broadcasted_iota(jnp.int32, (n, n), 1))
    xr = jnp.concatenate(
        [x_ref[:, q * PACK_H:(q + 1) * PACK_H] for q in range(QUARTERS)],
        axis=0).astype(jnp.bfloat16)
    o_ref[...] = jax.lax.dot_general(
        xr, eye, (((0,), (0,)), ((), ())),
        preferred_element_type=jnp.float32)


def _tc_pack(centers_t):
    return pl.pallas_call(
        _tc_pack_body,
        grid=(NUM_WINDOWS,),
        in_specs=[pl.BlockSpec((EMBED, PACK_W), lambda b: (0, b))],
        out_specs=pl.BlockSpec((PACK_H, ROW_W), lambda b: (b, 0)),
        out_shape=jax.ShapeDtypeStruct((TABLE_ROWS, ROW_W), jnp.float32),
    )(centers_t)


def _sc_gather(table, idx):
    num_steps = BATCH // GATHER_WINDOW
    mesh = plsc.VectorSubcoreMesh(core_axis_name="core",
                                  subcore_axis_name="subcore")

    @functools.partial(
        pl.kernel,
        out_type=jax.ShapeDtypeStruct((BATCH, ROW_W), table.dtype),
        mesh=mesh,
        compiler_params=pltpu.CompilerParams(use_tc_tiling_on_sc=True),
    )
    def gather_kernel(x_hbm, i_hbm, o_hbm):
        def body(i_vmem, o_vmem):
            pltpu.sync_copy(x_hbm.at[i_vmem.at[0]], o_vmem)

        pltpu.emit_pipeline(
            body,
            grid=(num_steps,),
            in_specs=[pl.BlockSpec((1, GATHER_WINDOW),
                                   index_map=lambda i: (0, i))],
            out_specs=[pl.BlockSpec((GATHER_WINDOW, ROW_W),
                                    index_map=lambda i: (i, 0))],
            core_axis_name=("core", "subcore"),
            dimension_semantics=(pltpu.PARALLEL,),
        )(i_hbm, o_hbm)

    return gather_kernel(table, idx.reshape(1, BATCH))


TC_BLOCK = 2048


def _tc_loss_body(ft_ref, g_ref, qf_ref, o_ref):
    n = EMBED + 1
    eye = jnp.float32(1.0) * (
        jax.lax.broadcasted_iota(jnp.int32, (n, n), 0)
        == jax.lax.broadcasted_iota(jnp.int32, (n, n), 1))
    xr = jnp.concatenate([ft_ref[...], qf_ref[...]], axis=0)
    fq = jax.lax.dot_general(
        xr, eye, (((0,), (0,)), ((), ())),
        precision=jax.lax.Precision.HIGHEST,
        preferred_element_type=jnp.float32)
    f = fq[:, :EMBED]
    q = fq[:, EMBED:]
    g = g_ref[...]
    c01 = jnp.where(q == 0.0, g[:, :EMBED], g[:, EMBED:2 * EMBED])
    c23 = jnp.where(q == 2.0, g[:, 2 * EMBED:3 * EMBED], g[:, 3 * EMBED:])
    c = jnp.where(q < 2.0, c01, c23)
    num = jnp.sum(f * c, axis=1, keepdims=True)
    ff = jnp.sum(f * f, axis=1, keepdims=True)
    cc = jnp.sum(c * c, axis=1, keepdims=True)
    cos = num * jax.lax.rsqrt(jnp.maximum(ff * cc, 1e-16))
    part = jnp.sum(1.0 - cos, axis=0, keepdims=True) / BATCH

    @pl.when(pl.program_id(0) == 0)
    def _():
        o_ref[...] = jnp.zeros_like(o_ref)

    o_ref[...] += part


def _tc_loss(features_t, gathered, quarter_row):
    return pl.pallas_call(
        _tc_loss_body,
        grid=(BATCH // TC_BLOCK,),
        in_specs=[
            pl.BlockSpec((EMBED, TC_BLOCK), lambda i: (0, i)),
            pl.BlockSpec((TC_BLOCK, ROW_W), lambda i: (i, 0)),
            pl.BlockSpec((1, TC_BLOCK), lambda i: (0, i)),
        ],
        out_specs=pl.BlockSpec((1, 1), lambda i: (0, 0)),
        out_shape=jax.ShapeDtypeStruct((1, 1), jnp.float32),
    )(features_t, gathered, quarter_row)


def kernel(features, labels, centers):
    labels32 = labels.astype(jnp.int32)
    table = _tc_pack(centers.T)
    w = labels32 // PACK_W
    r = labels32 % PACK_W
    slot = w * PACK_H + r % PACK_H
    quarter = (r // PACK_H).astype(jnp.float32)
    gathered = _sc_gather(table, slot)
    loss = _tc_loss(features.T, gathered, quarter.reshape(1, BATCH))
    return loss[0, 0]


# speedup vs baseline: 3.5252x; 1.1945x over previous
import functools

import jax
import jax.numpy as jnp
from jax.experimental import pallas as pl
from jax.experimental.pallas import tpu as pltpu
from jax.experimental.pallas import tpu_sc as plsc

BATCH = 16384
EMBED = 64
NUM_CLASSES = 1000000
QUARTERS = 4
PACK_H = 4096
PACK_W = QUARTERS * PACK_H
NUM_WINDOWS = (NUM_CLASSES + PACK_W - 1) // PACK_W
TABLE_ROWS = NUM_WINDOWS * PACK_H
ROW_W = 2 * EMBED
GATHER_WINDOW = 128


def _tc_pack_body(x_ref, o_ref):
    n = QUARTERS * EMBED
    eye = jnp.bfloat16(1.0) * (
        jax.lax.broadcasted_iota(jnp.int32, (n, n), 0)
        == jax.lax.broadcasted_iota(jnp.int32, (n, n), 1))
    xr = jnp.concatenate(
        [x_ref[:, q * PACK_H:(q + 1) * PACK_H] for q in range(QUARTERS)],
        axis=0).astype(jnp.bfloat16)
    t = jax.lax.dot_general(
        xr, eye, (((0,), (0,)), ((), ())),
        preferred_element_type=jnp.float32)
    tb = t.astype(jnp.bfloat16)
    lo = jax.lax.bitcast_convert_type(tb[:, :ROW_W], jnp.uint16)
    hi = jax.lax.bitcast_convert_type(tb[:, ROW_W:], jnp.uint16)
    packed = lo.astype(jnp.uint32) | (hi.astype(jnp.uint32) << 16)
    o_ref[...] = jax.lax.bitcast_convert_type(packed, jnp.int32)


def _tc_pack(centers_t):
    return pl.pallas_call(
        _tc_pack_body,
        grid=(NUM_WINDOWS,),
        in_specs=[pl.BlockSpec((EMBED, PACK_W), lambda b: (0, b))],
        out_specs=pl.BlockSpec((PACK_H, ROW_W), lambda b: (b, 0)),
        out_shape=jax.ShapeDtypeStruct((TABLE_ROWS, ROW_W), jnp.int32),
    )(centers_t)


def _sc_gather(table, idx):
    num_steps = BATCH // GATHER_WINDOW
    mesh = plsc.VectorSubcoreMesh(core_axis_name="core",
                                  subcore_axis_name="subcore")

    @functools.partial(
        pl.kernel,
        out_type=jax.ShapeDtypeStruct((BATCH, ROW_W), table.dtype),
        mesh=mesh,
        compiler_params=pltpu.CompilerParams(use_tc_tiling_on_sc=True),
    )
    def gather_kernel(x_hbm, i_hbm, o_hbm):
        def body(i_vmem, o_vmem):
            pltpu.sync_copy(x_hbm.at[i_vmem.at[0]], o_vmem)

        pltpu.emit_pipeline(
            body,
            grid=(num_steps,),
            in_specs=[pl.BlockSpec((1, GATHER_WINDOW),
                                   index_map=lambda i: (0, i))],
            out_specs=[pl.BlockSpec((GATHER_WINDOW, ROW_W),
                                    index_map=lambda i: (i, 0))],
            core_axis_name=("core", "subcore"),
            dimension_semantics=(pltpu.PARALLEL,),
        )(i_hbm, o_hbm)

    return gather_kernel(table, idx.reshape(1, BATCH))


TC_BLOCK = 2048


def _tc_loss_body(ft_ref, g_ref, qf_ref, o_ref):
    n = EMBED + 1
    eye = jnp.float32(1.0) * (
        jax.lax.broadcasted_iota(jnp.int32, (n, n), 0)
        == jax.lax.broadcasted_iota(jnp.int32, (n, n), 1))
    xr = jnp.concatenate([ft_ref[...], qf_ref[...]], axis=0)
    fq = jax.lax.dot_general(
        xr, eye, (((0,), (0,)), ((), ())),
        precision=jax.lax.Precision.HIGHEST,
        preferred_element_type=jnp.float32)
    f = fq[:, :EMBED]
    q = fq[:, EMBED:]
    gp = jax.lax.bitcast_convert_type(g_ref[...], jnp.uint32)
    ghalf = jnp.where(q < 2.0, gp & jnp.uint32(0xFFFF), gp >> 16)
    gb = jax.lax.bitcast_convert_type(ghalf.astype(jnp.uint16), jnp.bfloat16)
    c = jnp.where(q % 2.0 == 0.0, gb[:, :EMBED], gb[:, EMBED:])
    c = c.astype(jnp.float32)
    num = jnp.sum(f * c, axis=1, keepdims=True)
    ff = jnp.sum(f * f, axis=1, keepdims=True)
    cc = jnp.sum(c * c, axis=1, keepdims=True)
    cos = num * jax.lax.rsqrt(jnp.maximum(ff * cc, 1e-16))
    part = jnp.sum(1.0 - cos, axis=0, keepdims=True) / BATCH

    @pl.when(pl.program_id(0) == 0)
    def _():
        o_ref[...] = jnp.zeros_like(o_ref)

    o_ref[...] += part


def _tc_loss(features_t, gathered, quarter_row):
    return pl.pallas_call(
        _tc_loss_body,
        grid=(BATCH // TC_BLOCK,),
        in_specs=[
            pl.BlockSpec((EMBED, TC_BLOCK), lambda i: (0, i)),
            pl.BlockSpec((TC_BLOCK, ROW_W), lambda i: (i, 0)),
            pl.BlockSpec((1, TC_BLOCK), lambda i: (0, i)),
        ],
        out_specs=pl.BlockSpec((1, 1), lambda i: (0, 0)),
        out_shape=jax.ShapeDtypeStruct((1, 1), jnp.float32),
    )(features_t, gathered, quarter_row)


def kernel(features, labels, centers):
    labels32 = labels.astype(jnp.int32)
    table = _tc_pack(centers.T)
    w = labels32 // PACK_W
    r = labels32 % PACK_W
    slot = w * PACK_H + r % PACK_H
    quarter = (r // PACK_H).astype(jnp.float32)
    gathered = _sc_gather(table, slot)
    loss = _tc_loss(features.T, gathered, quarter.reshape(1, BATCH))
    return loss[0, 0]


# speedup vs baseline: 3.9670x; 1.1253x over previous
import functools

import jax
import jax.numpy as jnp
from jax.experimental import pallas as pl
from jax.experimental.pallas import tpu as pltpu
from jax.experimental.pallas import tpu_sc as plsc

BATCH = 16384
EMBED = 64
NUM_CLASSES = 1000000
QUARTERS = 4
PACK_H = 8192
PACK_W = QUARTERS * PACK_H
NUM_WINDOWS = (NUM_CLASSES + PACK_W - 1) // PACK_W
TABLE_ROWS = NUM_WINDOWS * PACK_H
ROW_W = 2 * EMBED
GATHER_WINDOW = 128


def _tc_pack_body(x_ref, o_ref):
    n = QUARTERS * EMBED
    eye = jnp.bfloat16(1.0) * (
        jax.lax.broadcasted_iota(jnp.int32, (n, n), 0)
        == jax.lax.broadcasted_iota(jnp.int32, (n, n), 1))
    xr = jnp.concatenate(
        [x_ref[:, q * PACK_H:(q + 1) * PACK_H] for q in range(QUARTERS)],
        axis=0).astype(jnp.bfloat16)
    t = jax.lax.dot_general(
        xr, eye, (((0,), (0,)), ((), ())),
        preferred_element_type=jnp.float32)
    tb = t.astype(jnp.bfloat16)
    lo = jax.lax.bitcast_convert_type(tb[:, :ROW_W], jnp.uint16)
    hi = jax.lax.bitcast_convert_type(tb[:, ROW_W:], jnp.uint16)
    packed = lo.astype(jnp.uint32) | (hi.astype(jnp.uint32) << 16)
    o_ref[...] = jax.lax.bitcast_convert_type(packed, jnp.int32)


def _tc_pack(centers_t):
    return pl.pallas_call(
        _tc_pack_body,
        grid=(NUM_WINDOWS,),
        in_specs=[pl.BlockSpec((EMBED, PACK_W), lambda b: (0, b))],
        out_specs=pl.BlockSpec((PACK_H, ROW_W), lambda b: (b, 0)),
        out_shape=jax.ShapeDtypeStruct((TABLE_ROWS, ROW_W), jnp.int32),
    )(centers_t)


def _sc_gather(table, idx):
    num_steps = BATCH // GATHER_WINDOW
    mesh = plsc.VectorSubcoreMesh(core_axis_name="core",
                                  subcore_axis_name="subcore")

    @functools.partial(
        pl.kernel,
        out_type=jax.ShapeDtypeStruct((BATCH, ROW_W), table.dtype),
        mesh=mesh,
        compiler_params=pltpu.CompilerParams(use_tc_tiling_on_sc=True),
    )
    def gather_kernel(x_hbm, i_hbm, o_hbm):
        def body(i_vmem, o_vmem):
            pltpu.sync_copy(x_hbm.at[i_vmem.at[0]], o_vmem)

        pltpu.emit_pipeline(
            body,
            grid=(num_steps,),
            in_specs=[pl.BlockSpec((1, GATHER_WINDOW),
                                   index_map=lambda i: (0, i))],
            out_specs=[pl.BlockSpec((GATHER_WINDOW, ROW_W),
                                    index_map=lambda i: (i, 0))],
            core_axis_name=("core", "subcore"),
            dimension_semantics=(pltpu.PARALLEL,),
        )(i_hbm, o_hbm)

    return gather_kernel(table, idx.reshape(1, BATCH))


TC_BLOCK = 4096


def _tc_loss_body(ft_ref, g_ref, qf_ref, o_ref):
    n = EMBED + 1
    eye = jnp.bfloat16(1.0) * (
        jax.lax.broadcasted_iota(jnp.int32, (n, n), 0)
        == jax.lax.broadcasted_iota(jnp.int32, (n, n), 1))
    xr = jnp.concatenate([ft_ref[...], qf_ref[...]],
                         axis=0).astype(jnp.bfloat16)
    fq = jax.lax.dot_general(
        xr, eye, (((0,), (0,)), ((), ())),
        preferred_element_type=jnp.float32)
    f = fq[:, :EMBED]
    q = fq[:, EMBED:]
    gp = jax.lax.bitcast_convert_type(g_ref[...], jnp.uint32)
    ghalf = jnp.where(q < 2.0, gp & jnp.uint32(0xFFFF), gp >> 16)
    gb = jax.lax.bitcast_convert_type(ghalf.astype(jnp.uint16), jnp.bfloat16)
    c = jnp.where(q % 2.0 == 0.0, gb[:, :EMBED], gb[:, EMBED:])
    c = c.astype(jnp.float32)
    num = jnp.sum(f * c, axis=1, keepdims=True)
    ff = jnp.sum(f * f, axis=1, keepdims=True)
    cc = jnp.sum(c * c, axis=1, keepdims=True)
    cos = num * jax.lax.rsqrt(jnp.maximum(ff * cc, 1e-16))
    part = jnp.sum(1.0 - cos, axis=0, keepdims=True) / BATCH

    @pl.when(pl.program_id(0) == 0)
    def _():
        o_ref[...] = jnp.zeros_like(o_ref)

    o_ref[...] += part


def _tc_loss(features_t, gathered, quarter_row):
    return pl.pallas_call(
        _tc_loss_body,
        grid=(BATCH // TC_BLOCK,),
        in_specs=[
            pl.BlockSpec((EMBED, TC_BLOCK), lambda i: (0, i)),
            pl.BlockSpec((TC_BLOCK, ROW_W), lambda i: (i, 0)),
            pl.BlockSpec((1, TC_BLOCK), lambda i: (0, i)),
        ],
        out_specs=pl.BlockSpec((1, 1), lambda i: (0, 0)),
        out_shape=jax.ShapeDtypeStruct((1, 1), jnp.float32),
    )(features_t, gathered, quarter_row)


def kernel(features, labels, centers):
    labels32 = labels.astype(jnp.int32)
    table = _tc_pack(centers.T)
    w = labels32 // PACK_W
    r = labels32 % PACK_W
    slot = w * PACK_H + r % PACK_H
    quarter = (r // PACK_H).astype(jnp.float32)
    gathered = _sc_gather(table, slot)
    loss = _tc_loss(features.T, gathered, quarter.reshape(1, BATCH))
    return loss[0, 0]
